# baseline (speedup 1.0000x reference)
import functools

import jax
import jax.numpy as jnp
from jax import lax
from jax.experimental import pallas as pl
from jax.experimental.pallas import tpu as pltpu
from jax.experimental.pallas import tpu_sc as plsc

_N_NODES = 10000
_N_GRAPHS = 64
_DIM = 128
_IR0, _IR1 = 0.5, 5.0
_DEV_COEFF = 0.2
_ES0, _ES1 = 0.0178, 0.0356
_N_SHORT, _N_LONG = 10.0, 6.0
_HB = (-0.7, 0.0)
_ML = (-0.7, 0.0)
_HP = (0.5, 1.5)


_NW = 32
_CH = 640


def _pad_idx(idx, e_pad):
    return jnp.pad(idx.astype(jnp.int32), (0, e_pad - idx.shape[0]))


def _sc_gather_multi(pairs, e, out_dims):
    k_chunks = -(-e // (_NW * _CH))
    e_pad = _NW * _CH * k_chunks
    mesh = plsc.VectorSubcoreMesh(core_axis_name="c", subcore_axis_name="s")
    n = len(pairs)
    dset = sorted(set(out_dims))

    @functools.partial(
        pl.kernel,
        mesh=mesh,
        out_type=[jax.ShapeDtypeStruct((e_pad, d), jnp.float32)
                  for d in out_dims],
        scratch_types=[pltpu.VMEM((_CH,), jnp.int32)]
                      + [pltpu.VMEM((_CH, d), jnp.float32) for d in dset]
                      + [pltpu.SemaphoreType.DMA],
    )
    def _k(*refs):
        tables = refs[0:2 * n:2]
        idxs = refs[1:2 * n:2]
        outs = refs[2 * n:3 * n]
        idx_v = refs[3 * n]
        bufs = {d: refs[3 * n + 1 + i] for i, d in enumerate(dset)}
        sem = refs[3 * n + 1 + len(dset)]
        wid = lax.axis_index("s") * 2 + lax.axis_index("c")
        base = wid * (_CH * k_chunks)
        for c in range(k_chunks):
            off = base + c * _CH
            for j in range(n):
                rv = bufs[out_dims[j]]
                pltpu.sync_copy(idxs[j].at[pl.ds(off, _CH)], idx_v)
                pltpu.async_copy(tables[j].at[idx_v], rv, sem).wait()
                pltpu.sync_copy(rv, outs[j].at[pl.ds(off, _CH)])

    flat = []
    for t, i in pairs:
        flat.extend((t, i))
    return _k(*flat)


def _mm_body(x_ref, w_ref, b_ref, o_ref, *, act):
    y = jnp.dot(x_ref[...], w_ref[...], preferred_element_type=jnp.float32)
    y = y + b_ref[...][None, :]
    if act == "relu":
        y = jnp.maximum(y, 0.0)
    o_ref[...] = y


def _mm(x, w, b=None, act=None, bn=2000):
    n, k = x.shape
    f = w.shape[1]
    if b is None:
        b = jnp.zeros((f,), jnp.float32)
    return pl.pallas_call(
        functools.partial(_mm_body, act=act),
        grid=(n // bn,),
        in_specs=[
            pl.BlockSpec((bn, k), lambda i: (i, 0)),
            pl.BlockSpec((k, f), lambda i: (0, 0)),
            pl.BlockSpec((f,), lambda i: (0,)),
        ],
        out_specs=pl.BlockSpec((bn, f), lambda i: (i, 0)),
        out_shape=jax.ShapeDtypeStruct((n, f), jnp.float32),
    )(x, w, b)


def _gate_body(x_ref, m_ref, wt_ref, wb_ref, gb_ref, o_ref, *, relu_m):
    m = m_ref[...]
    if relu_m:
        m = jnp.maximum(m, 0.0)
    logit = (
        jnp.dot(x_ref[...], wt_ref[...], preferred_element_type=jnp.float32)
        + jnp.dot(m, wb_ref[...], preferred_element_type=jnp.float32)
        + gb_ref[0]
    )
    c = jax.nn.sigmoid(logit)
    o_ref[...] = c * x_ref[...] + (1.0 - c) * m


def _gate(x, m, gw, gb, relu_m, bn=2000):
    n, k = x.shape
    wt, wb = gw[:k], gw[k:]
    return pl.pallas_call(
        functools.partial(_gate_body, relu_m=relu_m),
        grid=(n // bn,),
        in_specs=[
            pl.BlockSpec((bn, k), lambda i: (i, 0)),
            pl.BlockSpec((bn, k), lambda i: (i, 0)),
            pl.BlockSpec((k, 1), lambda i: (0, 0)),
            pl.BlockSpec((k, 1), lambda i: (0, 0)),
            pl.BlockSpec((1,), lambda i: (0,)),
        ],
        out_specs=pl.BlockSpec((bn, k), lambda i: (i, 0)),
        out_shape=jax.ShapeDtypeStruct((n, k), jnp.float32),
    )(x, m, wt, wb, gb)


def _pair_body(ae_ref, be_ref, ad_ref, bd_ref, s0_ref, s1_ref,
               b1e_ref, w2e_ref, b2e_ref, b1d_ref, w2d_ref, b2d_ref,
               rotor_ref, cf_ref, en_ref, dv_ref):
    i = pl.program_id(0)
    nsteps = pl.num_programs(0)

    he = jnp.maximum(ae_ref[...] + be_ref[...] + b1e_ref[...][None, :], 0.0)
    eps_logit = jnp.dot(he, w2e_ref[...], preferred_element_type=jnp.float32) + b2e_ref[0]
    eps = jax.nn.sigmoid(eps_logit[:, 0]) * (_ES1 - _ES0) + _ES0

    hd = jnp.maximum(ad_ref[...] + bd_ref[...] + b1d_ref[...][None, :], 0.0)
    dv_logit = jnp.dot(hd, w2d_ref[...], preferred_element_type=jnp.float32) + b2d_ref[0]
    dvdw = jnp.tanh(dv_logit[:, 0]) * _DEV_COEFF

    s0 = s0_ref[...]
    s1 = s1_ref[...]
    dx = s0[:, 0] - s1[:, 0]
    dy = s0[:, 1] - s1[:, 1]
    dz = s0[:, 2] - s1[:, 2]
    D = jnp.sqrt(dx * dx + dy * dy + dz * dz + 1e-12)

    lig0, lig1 = s0[:, 4], s1[:, 4]
    met0, met1 = s0[:, 5], s1[:, 5]
    don0, don1 = s0[:, 6], s1[:, 6]
    acc0, acc1 = s0[:, 7], s1[:, 7]
    hyd0, hyd1 = s0[:, 8], s1[:, 8]
    bat0, bat1 = s0[:, 9], s1[:, 9]

    pair_ok = lig0 * (1.0 - lig1) * (bat0 == bat1).astype(jnp.float32)
    maskf = pair_ok * (D >= _IR0).astype(jnp.float32) * (D <= _IR1).astype(jnp.float32)

    R = s0[:, 3] + s1[:, 3] + dvdw
    Dc = jnp.maximum(D, _IR0)
    ratio = R / Dc
    lj = jnp.minimum(ratio ** _N_SHORT - 2.0 * ratio ** _N_LONG, 100.0) * eps

    hbc = cf_ref[0]
    hpc = cf_ref[1]
    rc = cf_ref[2]
    min_hb = -(hbc * hbc)
    min_hp = -(hpc * hpc)
    dev = Dc - R

    def _lp(minima, c0, c1):
        frac = jnp.clip((c1 - dev) / (c1 - c0), 0.0, 1.0)
        return minima * frac

    e_hb = _lp(min_hb, _HB[0], _HB[1])
    e_ml = _lp(min_hb, _ML[0], _ML[1])
    e_hp = _lp(min_hp, _HP[0], _HP[1])

    not_metal = (1.0 - met0) * (1.0 - met1)
    m_hb = jnp.minimum(don0 * acc1 + acc0 * don1, 1.0) * not_metal
    m_ml = jnp.minimum(met0 * acc1 + acc0 * met1, 1.0)
    m_hp = hyd0 * hyd1 * not_metal

    ep = jnp.stack(
        [lj * not_metal, e_hb * m_hb, e_ml * m_ml, e_hp * m_hp], axis=1
    ) * maskf[:, None]

    gi = jax.lax.broadcasted_iota(jnp.int32, (_N_GRAPHS, ep.shape[0]), 0)
    onehot = (gi == bat0.astype(jnp.int32)[None, :]).astype(jnp.float32)
    part = jnp.dot(onehot, ep, preferred_element_type=jnp.float32,
                   precision=jax.lax.Precision.HIGHEST)

    @pl.when(i == 0)
    def _():
        en_ref[...] = jnp.zeros_like(en_ref)

    en_ref[...] += part

    @pl.when(i == nsteps - 1)
    def _():
        penalty = 1.0 + rc * rc * rotor_ref[...]
        en_ref[...] = en_ref[...] / penalty

    dv_ref[...] = (dvdw * maskf)[None, None, :]


def _pair_stage(ae, be, ad, bd, s0, s1, pe, pd, rotor, coeffs, eb=2000):
    e = ae.shape[0]
    grid = e // eb
    en, dv = pl.pallas_call(
        _pair_body,
        grid=(grid,),
        in_specs=[
            pl.BlockSpec((eb, _DIM), lambda i: (i, 0)),
            pl.BlockSpec((eb, _DIM), lambda i: (i, 0)),
            pl.BlockSpec((eb, _DIM), lambda i: (i, 0)),
            pl.BlockSpec((eb, _DIM), lambda i: (i, 0)),
            pl.BlockSpec((eb, _DIM), lambda i: (i, 0)),
            pl.BlockSpec((eb, _DIM), lambda i: (i, 0)),
            pl.BlockSpec((_DIM,), lambda i: (0,)),
            pl.BlockSpec((_DIM, 1), lambda i: (0, 0)),
            pl.BlockSpec((1,), lambda i: (0,)),
            pl.BlockSpec((_DIM,), lambda i: (0,)),
            pl.BlockSpec((_DIM, 1), lambda i: (0, 0)),
            pl.BlockSpec((1,), lambda i: (0,)),
            pl.BlockSpec((_N_GRAPHS, 1), lambda i: (0, 0)),
            pl.BlockSpec((3,), lambda i: (0,)),
        ],
        out_specs=[
            pl.BlockSpec((_N_GRAPHS, 4), lambda i: (0, 0)),
            pl.BlockSpec((1, 1, eb), lambda i: (i, 0, 0)),
        ],
        out_shape=[
            jax.ShapeDtypeStruct((_N_GRAPHS, 4), jnp.float32),
            jax.ShapeDtypeStruct((grid, 1, eb), jnp.float32),
        ],
    )(ae, be, ad, bd, s0, s1, pe["b1"], pe["W2"], pe["b2"],
      pd["b1"], pd["W2"], pd["b2"], rotor, coeffs)
    return en, dv.reshape(e)


def kernel(x, edge_index, edge_index_c, edge_index_i, pos, vdw_radii, batch,
           is_ligand, is_metal, is_h_donor, is_h_acceptor, is_hydrophobic,
           rotor, params):
    h = _mm(x, params["embed_W"])

    src, dst = edge_index[0], edge_index[1]
    e_intra = src.shape[0]
    ep_intra = _NW * _CH * (-(-e_intra // (_NW * _CH)))
    src_p = _pad_idx(src, ep_intra)
    dst_p = _pad_idx(dst, ep_intra)
    ecnt = jnp.maximum(
        jax.ops.segment_sum(jnp.ones((e_intra,), jnp.float32), dst,
                            num_segments=_N_NODES), 1.0)
    for p in params["gat"]:
        h1 = _mm(h, p["W"], p["b"])
        hA = _mm(h1, p["A"])
        gas, g1d, gad, g1s = _sc_gather_multi(
            [(hA, src_p), (h1, dst_p), (hA, dst_p), (h1, src_p)],
            e_intra, [_DIM, _DIM, _DIM, _DIM])
        e = jnp.sum(gas[:e_intra] * g1d[:e_intra]
                    + gad[:e_intra] * g1s[:e_intra], -1)
        e_mean = jax.ops.segment_sum(e, dst, num_segments=_N_NODES) / ecnt
        w = jnp.exp(jnp.minimum(e - e_mean[dst], 80.0))
        denom = jax.ops.segment_sum(w, dst, num_segments=_N_NODES)
        attn = w / (denom[dst] + 1e-16)
        hp = jax.ops.segment_sum(attn[:, None] * g1s[:e_intra], dst,
                                 num_segments=_N_NODES)
        h = _gate(h, hp, p["gW"], p["gb"], relu_m=True)

    srcc, dstc = edge_index_c[0], edge_index_c[1]
    e_inter = srcc.shape[0]
    ep_inter = _NW * _CH * (-(-e_inter // (_NW * _CH)))
    srcc_p = _pad_idx(srcc, ep_inter)
    for p in params["inter"]:
        hr = _mm(h, p["W"], p["b"], act="relu")
        (ghr,) = _sc_gather_multi([(hr, srcc_p)], e_inter, [_DIM])
        m = jax.ops.segment_sum(ghr[:e_inter], dstc, num_segments=_N_NODES)
        h = _gate(h, m, p["gW"], p["gb"], relu_m=False)

    pe, pd = params["vdw_eps"], params["dvdw"]
    g1 = _mm(h, pe["W1"][:_DIM])
    g2 = _mm(h, pe["W1"][_DIM:])
    g3 = _mm(h, pd["W1"][:_DIM])
    g4 = _mm(h, pd["W1"][_DIM:])

    scal = jnp.concatenate(
        [
            pos,
            vdw_radii[:, None],
            is_ligand[:, None].astype(jnp.float32),
            is_metal[:, None].astype(jnp.float32),
            is_h_donor[:, None].astype(jnp.float32),
            is_h_acceptor[:, None].astype(jnp.float32),
            is_hydrophobic[:, None].astype(jnp.float32),
            batch[:, None].astype(jnp.float32),
            jnp.zeros((_N_NODES, _DIM - 10), jnp.float32),
        ],
        axis=1,
    )

    i0, i1 = edge_index_i[0], edge_index_i[1]
    e_pair = i0.shape[0]
    ep_pair = _NW * _CH * (-(-e_pair // (_NW * _CH)))
    i0_p = _pad_idx(i0, ep_pair)
    i1_p = _pad_idx(i1, ep_pair)
    ae, be, ad, bd, s0, s1 = _sc_gather_multi(
        [(g1, i0_p), (g2, i1_p), (g3, i0_p), (g4, i1_p),
         (scal, i0_p), (scal, i1_p)],
        e_pair, [_DIM, _DIM, _DIM, _DIM, _DIM, _DIM])
    ae, be, ad, bd = (ae[:e_pair], be[:e_pair], ad[:e_pair], bd[:e_pair])
    s0, s1 = s0[:e_pair], s1[:e_pair]

    coeffs = jnp.concatenate(
        [params["hbond_coeff"], params["hydrophobic_coeff"], params["rotor_coeff"]]
    )
    energies, dvdw_masked = _pair_stage(ae, be, ad, bd, s0, s1, pe, pd,
                                        rotor, coeffs)
    return energies, dvdw_masked


# speedup vs baseline: 1.4083x; 1.4083x over previous
import functools

import jax
import jax.numpy as jnp
from jax import lax
from jax.experimental import pallas as pl
from jax.experimental.pallas import tpu as pltpu
from jax.experimental.pallas import tpu_sc as plsc

_N_NODES = 10000
_N_GRAPHS = 64
_DIM = 128
_IR0, _IR1 = 0.5, 5.0
_DEV_COEFF = 0.2
_ES0, _ES1 = 0.0178, 0.0356
_N_SHORT, _N_LONG = 10.0, 6.0
_HB = (-0.7, 0.0)
_ML = (-0.7, 0.0)
_HP = (0.5, 1.5)


_NW = 32
_CH = 640


def _pad_idx(idx, e_pad):
    return jnp.pad(idx.astype(jnp.int32), (0, e_pad - idx.shape[0]))


def _sc_gather_multi(pairs, e, out_dims):
    k_chunks = -(-e // (_NW * _CH))
    e_pad = _NW * _CH * k_chunks
    mesh = plsc.VectorSubcoreMesh(core_axis_name="c", subcore_axis_name="s")
    n = len(pairs)
    dset = sorted(set(out_dims))

    @functools.partial(
        pl.kernel,
        mesh=mesh,
        out_type=[jax.ShapeDtypeStruct((e_pad, d) if d else (e_pad,),
                                       jnp.float32)
                  for d in out_dims],
        scratch_types=[pltpu.VMEM((_CH,), jnp.int32)]
                      + [pltpu.VMEM((_CH, d) if d else (_CH,), jnp.float32)
                         for d in dset]
                      + [pltpu.SemaphoreType.DMA],
    )
    def _k(*refs):
        tables = refs[0:2 * n:2]
        idxs = refs[1:2 * n:2]
        outs = refs[2 * n:3 * n]
        idx_v = refs[3 * n]
        bufs = {d: refs[3 * n + 1 + i] for i, d in enumerate(dset)}
        sem = refs[3 * n + 1 + len(dset)]
        wid = lax.axis_index("s") * 2 + lax.axis_index("c")
        base = wid * (_CH * k_chunks)
        for c in range(k_chunks):
            off = base + c * _CH
            for j in range(n):
                rv = bufs[out_dims[j]]
                pltpu.sync_copy(idxs[j].at[pl.ds(off, _CH)], idx_v)
                pltpu.async_copy(tables[j].at[idx_v], rv, sem).wait()
                pltpu.sync_copy(rv, outs[j].at[pl.ds(off, _CH)])

    flat = []
    for t, i in pairs:
        flat.extend((t, i))
    return _k(*flat)


def _mm_body(x_ref, w_ref, b_ref, o_ref, *, act):
    y = jnp.dot(x_ref[...], w_ref[...], preferred_element_type=jnp.float32)
    y = y + b_ref[...][None, :]
    if act == "relu":
        y = jnp.maximum(y, 0.0)
    o_ref[...] = y


def _mm(x, w, b=None, act=None, bn=2000):
    n, k = x.shape
    f = w.shape[1]
    if b is None:
        b = jnp.zeros((f,), jnp.float32)
    return pl.pallas_call(
        functools.partial(_mm_body, act=act),
        grid=(n // bn,),
        in_specs=[
            pl.BlockSpec((bn, k), lambda i: (i, 0)),
            pl.BlockSpec((k, f), lambda i: (0, 0)),
            pl.BlockSpec((f,), lambda i: (0,)),
        ],
        out_specs=pl.BlockSpec((bn, f), lambda i: (i, 0)),
        out_shape=jax.ShapeDtypeStruct((n, f), jnp.float32),
    )(x, w, b)


def _gate_body(x_ref, m_ref, wt_ref, wb_ref, gb_ref, den_ref, o_ref, *, relu_m,
               use_den):
    m = m_ref[...]
    if use_den:
        m = m / (den_ref[...] + 1e-16)
    if relu_m:
        m = jnp.maximum(m, 0.0)
    logit = (
        jnp.dot(x_ref[...], wt_ref[...], preferred_element_type=jnp.float32)
        + jnp.dot(m, wb_ref[...], preferred_element_type=jnp.float32)
        + gb_ref[0]
    )
    c = jax.nn.sigmoid(logit)
    o_ref[...] = c * x_ref[...] + (1.0 - c) * m


def _gate(x, m, gw, gb, relu_m, denom=None, bn=2000):
    n, k = x.shape
    wt, wb = gw[:k], gw[k:]
    use_den = denom is not None
    if denom is None:
        denom = jnp.ones((n, 1), jnp.float32)
    return pl.pallas_call(
        functools.partial(_gate_body, relu_m=relu_m, use_den=use_den),
        grid=(n // bn,),
        in_specs=[
            pl.BlockSpec((bn, k), lambda i: (i, 0)),
            pl.BlockSpec((bn, k), lambda i: (i, 0)),
            pl.BlockSpec((k, 1), lambda i: (0, 0)),
            pl.BlockSpec((k, 1), lambda i: (0, 0)),
            pl.BlockSpec((1,), lambda i: (0,)),
            pl.BlockSpec((bn, 1), lambda i: (i, 0)),
        ],
        out_specs=pl.BlockSpec((bn, k), lambda i: (i, 0)),
        out_shape=jax.ShapeDtypeStruct((n, k), jnp.float32),
    )(x, m, wt, wb, gb, denom)


def _pair_body(ae_ref, be_ref, ad_ref, bd_ref, s0_ref, s1_ref,
               b1e_ref, w2e_ref, b2e_ref, b1d_ref, w2d_ref, b2d_ref,
               rotor_ref, cf_ref, en_ref, dv_ref):
    i = pl.program_id(0)
    nsteps = pl.num_programs(0)

    he = jnp.maximum(ae_ref[...] + be_ref[...] + b1e_ref[...][None, :], 0.0)
    eps_logit = jnp.dot(he, w2e_ref[...], preferred_element_type=jnp.float32) + b2e_ref[0]
    eps = jax.nn.sigmoid(eps_logit[:, 0]) * (_ES1 - _ES0) + _ES0

    hd = jnp.maximum(ad_ref[...] + bd_ref[...] + b1d_ref[...][None, :], 0.0)
    dv_logit = jnp.dot(hd, w2d_ref[...], preferred_element_type=jnp.float32) + b2d_ref[0]
    dvdw = jnp.tanh(dv_logit[:, 0]) * _DEV_COEFF

    s0 = s0_ref[...]
    s1 = s1_ref[...]
    dx = s0[:, 0] - s1[:, 0]
    dy = s0[:, 1] - s1[:, 1]
    dz = s0[:, 2] - s1[:, 2]
    D = jnp.sqrt(dx * dx + dy * dy + dz * dz + 1e-12)

    lig0, lig1 = s0[:, 4], s1[:, 4]
    met0, met1 = s0[:, 5], s1[:, 5]
    don0, don1 = s0[:, 6], s1[:, 6]
    acc0, acc1 = s0[:, 7], s1[:, 7]
    hyd0, hyd1 = s0[:, 8], s1[:, 8]
    bat0, bat1 = s0[:, 9], s1[:, 9]

    pair_ok = lig0 * (1.0 - lig1) * (bat0 == bat1).astype(jnp.float32)
    maskf = pair_ok * (D >= _IR0).astype(jnp.float32) * (D <= _IR1).astype(jnp.float32)

    R = s0[:, 3] + s1[:, 3] + dvdw
    Dc = jnp.maximum(D, _IR0)
    ratio = R / Dc
    lj = jnp.minimum(ratio ** _N_SHORT - 2.0 * ratio ** _N_LONG, 100.0) * eps

    hbc = cf_ref[0]
    hpc = cf_ref[1]
    rc = cf_ref[2]
    min_hb = -(hbc * hbc)
    min_hp = -(hpc * hpc)
    dev = Dc - R

    def _lp(minima, c0, c1):
        frac = jnp.clip((c1 - dev) / (c1 - c0), 0.0, 1.0)
        return minima * frac

    e_hb = _lp(min_hb, _HB[0], _HB[1])
    e_ml = _lp(min_hb, _ML[0], _ML[1])
    e_hp = _lp(min_hp, _HP[0], _HP[1])

    not_metal = (1.0 - met0) * (1.0 - met1)
    m_hb = jnp.minimum(don0 * acc1 + acc0 * don1, 1.0) * not_metal
    m_ml = jnp.minimum(met0 * acc1 + acc0 * met1, 1.0)
    m_hp = hyd0 * hyd1 * not_metal

    ep = jnp.stack(
        [lj * not_metal, e_hb * m_hb, e_ml * m_ml, e_hp * m_hp], axis=1
    ) * maskf[:, None]

    gi = jax.lax.broadcasted_iota(jnp.int32, (_N_GRAPHS, ep.shape[0]), 0)
    onehot = (gi == bat0.astype(jnp.int32)[None, :]).astype(jnp.float32)
    part = jnp.dot(onehot, ep, preferred_element_type=jnp.float32,
                   precision=jax.lax.Precision.HIGHEST)

    @pl.when(i == 0)
    def _():
        en_ref[...] = jnp.zeros_like(en_ref)

    en_ref[...] += part

    @pl.when(i == nsteps - 1)
    def _():
        penalty = 1.0 + rc * rc * rotor_ref[...]
        en_ref[...] = en_ref[...] / penalty

    dv_ref[...] = (dvdw * maskf)[None, None, :]


def _pair_stage(ae, be, ad, bd, s0, s1, pe, pd, rotor, coeffs, eb=2000):
    e = ae.shape[0]
    grid = e // eb
    en, dv = pl.pallas_call(
        _pair_body,
        grid=(grid,),
        in_specs=[
            pl.BlockSpec((eb, _DIM), lambda i: (i, 0)),
            pl.BlockSpec((eb, _DIM), lambda i: (i, 0)),
            pl.BlockSpec((eb, _DIM), lambda i: (i, 0)),
            pl.BlockSpec((eb, _DIM), lambda i: (i, 0)),
            pl.BlockSpec((eb, _DIM), lambda i: (i, 0)),
            pl.BlockSpec((eb, _DIM), lambda i: (i, 0)),
            pl.BlockSpec((_DIM,), lambda i: (0,)),
            pl.BlockSpec((_DIM, 1), lambda i: (0, 0)),
            pl.BlockSpec((1,), lambda i: (0,)),
            pl.BlockSpec((_DIM,), lambda i: (0,)),
            pl.BlockSpec((_DIM, 1), lambda i: (0, 0)),
            pl.BlockSpec((1,), lambda i: (0,)),
            pl.BlockSpec((_N_GRAPHS, 1), lambda i: (0, 0)),
            pl.BlockSpec((3,), lambda i: (0,)),
        ],
        out_specs=[
            pl.BlockSpec((_N_GRAPHS, 4), lambda i: (0, 0)),
            pl.BlockSpec((1, 1, eb), lambda i: (i, 0, 0)),
        ],
        out_shape=[
            jax.ShapeDtypeStruct((_N_GRAPHS, 4), jnp.float32),
            jax.ShapeDtypeStruct((grid, 1, eb), jnp.float32),
        ],
    )(ae, be, ad, bd, s0, s1, pe["b1"], pe["W2"], pe["b2"],
      pd["b1"], pd["W2"], pd["b2"], rotor, coeffs)
    return en, dv.reshape(e)


def kernel(x, edge_index, edge_index_c, edge_index_i, pos, vdw_radii, batch,
           is_ligand, is_metal, is_h_donor, is_h_acceptor, is_hydrophobic,
           rotor, params):
    h = _mm(x, params["embed_W"])

    src, dst = edge_index[0], edge_index[1]
    e_intra = src.shape[0]
    ep_intra = _NW * _CH * (-(-e_intra // (_NW * _CH)))
    src_p = _pad_idx(src, ep_intra)
    dst_p = _pad_idx(dst, ep_intra)
    ecnt = jnp.maximum(
        jax.ops.segment_sum(jnp.ones((e_intra,), jnp.float32), dst,
                            num_segments=_N_NODES), 1.0)
    for p in params["gat"]:
        h1 = _mm(h, p["W"], p["b"])
        hA = _mm(h1, p["A"])
        gas, g1d, gad, g1s = _sc_gather_multi(
            [(hA, src_p), (h1, dst_p), (hA, dst_p), (h1, src_p)],
            e_intra, [_DIM, _DIM, _DIM, _DIM])
        e = jnp.sum(gas[:e_intra] * g1d[:e_intra]
                    + gad[:e_intra] * g1s[:e_intra], -1)
        e_mean = jax.ops.segment_sum(e, dst, num_segments=_N_NODES) / ecnt
        (em_e,) = _sc_gather_multi([(e_mean, dst_p)], e_intra, [0])
        w = jnp.exp(jnp.minimum(e - em_e[:e_intra], 80.0))
        denom = jax.ops.segment_sum(w, dst, num_segments=_N_NODES)
        u = jax.ops.segment_sum(w[:, None] * g1s[:e_intra], dst,
                                num_segments=_N_NODES)
        h = _gate(h, u, p["gW"], p["gb"], relu_m=True, denom=denom[:, None])

    srcc, dstc = edge_index_c[0], edge_index_c[1]
    e_inter = srcc.shape[0]
    ep_inter = _NW * _CH * (-(-e_inter // (_NW * _CH)))
    srcc_p = _pad_idx(srcc, ep_inter)
    for p in params["inter"]:
        hr = _mm(h, p["W"], p["b"], act="relu")
        (ghr,) = _sc_gather_multi([(hr, srcc_p)], e_inter, [_DIM])
        m = jax.ops.segment_sum(ghr[:e_inter], dstc, num_segments=_N_NODES)
        h = _gate(h, m, p["gW"], p["gb"], relu_m=False)

    pe, pd = params["vdw_eps"], params["dvdw"]
    g1 = _mm(h, pe["W1"][:_DIM])
    g2 = _mm(h, pe["W1"][_DIM:])
    g3 = _mm(h, pd["W1"][:_DIM])
    g4 = _mm(h, pd["W1"][_DIM:])

    scal = jnp.concatenate(
        [
            pos,
            vdw_radii[:, None],
            is_ligand[:, None].astype(jnp.float32),
            is_metal[:, None].astype(jnp.float32),
            is_h_donor[:, None].astype(jnp.float32),
            is_h_acceptor[:, None].astype(jnp.float32),
            is_hydrophobic[:, None].astype(jnp.float32),
            batch[:, None].astype(jnp.float32),
            jnp.zeros((_N_NODES, _DIM - 10), jnp.float32),
        ],
        axis=1,
    )

    i0, i1 = edge_index_i[0], edge_index_i[1]
    e_pair = i0.shape[0]
    ep_pair = _NW * _CH * (-(-e_pair // (_NW * _CH)))
    i0_p = _pad_idx(i0, ep_pair)
    i1_p = _pad_idx(i1, ep_pair)
    ae, be, ad, bd, s0, s1 = _sc_gather_multi(
        [(g1, i0_p), (g2, i1_p), (g3, i0_p), (g4, i1_p),
         (scal, i0_p), (scal, i1_p)],
        e_pair, [_DIM, _DIM, _DIM, _DIM, _DIM, _DIM])
    ae, be, ad, bd = (ae[:e_pair], be[:e_pair], ad[:e_pair], bd[:e_pair])
    s0, s1 = s0[:e_pair], s1[:e_pair]

    coeffs = jnp.concatenate(
        [params["hbond_coeff"], params["hydrophobic_coeff"], params["rotor_coeff"]]
    )
    energies, dvdw_masked = _pair_stage(ae, be, ad, bd, s0, s1, pe, pd,
                                        rotor, coeffs)
    return energies, dvdw_masked
